# baseline (speedup 1.0000x reference)
import functools

import numpy as np
import jax
import jax.numpy as jnp
from jax import lax
from jax.experimental import pallas as pl
from jax.experimental.pallas import tpu as pltpu
from jax.experimental.pallas import tpu_sc as plsc

_LEVELS = 256
_D = 8192
_B = 4096
_NW = 32
_B_SC = 3072
_B_TC = _B - _B_SC
_ROWS_PER_W = _B_SC // _NW
_WPR = _D // 32
_G = 8
_NGROUPS = _ROWS_PER_W // _G
_HALF = _G // 2


def _prep(x, w):
    def body(x_ref, w_ref, pk_ref, idx_ref):
        wv = w_ref[...]
        bits = lax.shift_right_logical(
            lax.bitcast_convert_type(wv, jnp.int32), 31)
        word = bits[:, 0:_WPR]
        for k in range(1, 32):
            word = word | (bits[:, k * _WPR:(k + 1) * _WPR] << k)
        pk_ref[pl.ds(2 * _LEVELS, _LEVELS), :] = word
        wrap = word[:, _WPR - 2:]
        wrap = lax.shift_left(wrap, 1) | lax.shift_right_logical(wrap, 31)
        pk_ref[pl.ds(0, _LEVELS), :] = jnp.concatenate(
            [wrap, word[:, :_WPR - 2]], axis=1)
        pk_ref[pl.ds(_LEVELS, _LEVELS), :] = jnp.concatenate(
            [wrap[:, 1:], word[:, :_WPR - 1]], axis=1)

        xv = x_ref[...]
        q = jnp.clip(jnp.round(xv * (_LEVELS - 1)).astype(jnp.int32),
                     0, _LEVELS - 1)
        ch = lax.broadcasted_iota(jnp.int32, (_B, 3), 1)
        idx_ref[...] = q + ch * _LEVELS

    pk, idx = pl.pallas_call(
        body,
        out_shape=[
            jax.ShapeDtypeStruct((3 * _LEVELS, _WPR), jnp.int32),
            jax.ShapeDtypeStruct((_B, 3), jnp.int32),
        ],
    )(x, w)
    return pk, idx


def _sc_encode(pk_tab, idx_flat):
    mesh = plsc.VectorSubcoreMesh(
        core_axis_name="c", subcore_axis_name="s", num_cores=2, num_subcores=16)

    @functools.partial(
        pl.kernel,
        mesh=mesh,
        out_type=jax.ShapeDtypeStruct((_B, _D), jnp.float32),
        scratch_types=[
            pltpu.VMEM((_ROWS_PER_W * 3,), jnp.int32),
            pltpu.VMEM((2, 3 * _G, _WPR), jnp.int32),
            pltpu.VMEM((2, _HALF, _D), jnp.float32),
            pltpu.SemaphoreType.DMA,
            pltpu.SemaphoreType.DMA,
            pltpu.SemaphoreType.DMA,
        ],
    )
    def k(pk_hbm, idx_hbm, out_hbm, idx_v, gbuf, obuf, gsem, osem0, osem1):
        wid = lax.axis_index("s") * 2 + lax.axis_index("c")
        base = wid * _ROWS_PER_W
        pltpu.sync_copy(idx_hbm.at[pl.ds(base * 3, _ROWS_PER_W * 3)], idx_v)

        def start_gather(g, par):
            pltpu.async_copy(
                pk_hbm.at[idx_v.at[pl.ds(g * (3 * _G), 3 * _G)]],
                gbuf.at[par], gsem)

        start_gather(0, 0)

        sign_mask = jnp.full((16,), np.int32(-2147483648), jnp.int32)
        one_bits = jnp.full((16,), np.int32(0x3F800000), jnp.int32)

        def group(g, carry):
            par = lax.bitwise_and(g, 1)
            pltpu.make_async_copy(
                pk_hbm.at[idx_v.at[pl.ds(0, 3 * _G)]],
                gbuf.at[par], gsem).wait()

            @pl.when(g + 1 < _NGROUPS)
            def _():
                start_gather(g + 1, 1 - par)

            for h, osem in ((0, osem0), (1, osem1)):
                @pl.when(g > 0)
                def _(h=h, osem=osem):
                    pltpu.make_async_copy(
                        obuf.at[h],
                        out_hbm.at[pl.ds(base, _HALF)], osem).wait()

                def row(pp, c2, h=h):
                    p = h * _HALF + pp

                    def wordblk(wi, c3, p=p):
                        a = gbuf[par, 3 * p, pl.ds(wi * 16, 16)]
                        b = gbuf[par, 3 * p + 1, pl.ds(wi * 16, 16)]
                        d = gbuf[par, 3 * p + 2, pl.ds(wi * 16, 16)]
                        wv = lax.bitwise_xor(lax.bitwise_xor(a, b), d)
                        for kk in range(32):
                            s = lax.bitwise_and(
                                lax.shift_left(wv, jnp.full((16,), 31 - kk,
                                                            jnp.int32)),
                                sign_mask)
                            val = lax.bitcast_convert_type(
                                lax.bitwise_or(s, one_bits), jnp.float32)
                            obuf[h, pp, pl.ds(kk * _WPR + wi * 16, 16)] = val
                        return c3

                    lax.fori_loop(0, _WPR // 16, wordblk, 0)
                    return c2

                lax.fori_loop(0, _HALF, row, 0)
                pltpu.async_copy(
                    obuf.at[h],
                    out_hbm.at[pl.ds(base + g * _G + h * _HALF, _HALF)], osem)
            return carry

        lax.fori_loop(0, _NGROUPS, group, 0)
        for h, osem in ((0, osem0), (1, osem1)):
            pltpu.make_async_copy(
                obuf.at[h], out_hbm.at[pl.ds(base, _HALF)], osem).wait()

    return k(pk_tab, idx_flat)


def _tc_onehot(pk, idx):
    sign_mask = np.int32(-2147483648)
    one_bits = np.int32(0x3F800000)

    blk = 256

    def body(pk_ref, idx_ref, o_ref, bits_ref):
        @pl.when(pl.program_id(0) == 0)
        def _():
            words = pk_ref[...]
            planes = []
            for k in range(32):
                planes.append(lax.shift_right_logical(
                    lax.shift_left(words, 31 - k), 31).astype(jnp.bfloat16))
            bits_ref[...] = jnp.concatenate(planes, axis=1)

        idxv = idx_ref[...]
        e = lax.broadcasted_iota(jnp.int32, (blk, 3 * _LEVELS), 1)
        oh = ((e == idxv[:, 0:1]) | (e == idxv[:, 1:2])
              | (e == idxv[:, 2:3])).astype(jnp.bfloat16)
        s = jnp.dot(oh, bits_ref[...], preferred_element_type=jnp.float32)
        si = s.astype(jnp.int32)
        o_ref[...] = lax.bitcast_convert_type(
            lax.bitwise_or(lax.bitwise_and(lax.shift_left(si, 31), sign_mask),
                           one_bits), jnp.float32)

    return pl.pallas_call(
        body,
        grid=(_B_TC // blk,),
        in_specs=[
            pl.BlockSpec((3 * _LEVELS, _WPR), lambda i: (0, 0)),
            pl.BlockSpec((blk, 3), lambda i: (i, 0)),
        ],
        out_specs=pl.BlockSpec((blk, _D), lambda i: (i, 0)),
        out_shape=jax.ShapeDtypeStruct((_B_TC, _D), jnp.float32),
        scratch_shapes=[pltpu.VMEM((3 * _LEVELS, _D), jnp.bfloat16)],
    )(pk, idx[_B_SC:])


def kernel(x, level_weight):
    pk_tab, idx = _prep(x, level_weight)
    sc_out = _sc_encode(pk_tab, idx.reshape(-1)[: _B_SC * 3])
    tc_out = _tc_onehot(pk_tab, idx)
    return lax.dynamic_update_slice(sc_out, tc_out, (_B_SC, 0))


# speedup vs baseline: 1.1786x; 1.1786x over previous
import functools

import numpy as np
import jax
import jax.numpy as jnp
from jax import lax
from jax.experimental import pallas as pl
from jax.experimental.pallas import tpu as pltpu
from jax.experimental.pallas import tpu_sc as plsc

_LEVELS = 256
_D = 8192
_B = 4096
_NW = 32
_ROWS_PER_W = _B // _NW
_WPR = _D // 32
_G = 8
_NGROUPS = _ROWS_PER_W // _G
_EXP_ROWS = 512


def _prep(x, w):
    def body(x_ref, w_ref, pk_ref, idx_ref):
        wv = w_ref[...]
        bits = lax.shift_right_logical(
            lax.bitcast_convert_type(wv, jnp.int32), 31)
        word = bits[:, 0:_WPR]
        for k in range(1, 32):
            word = word | (bits[:, k * _WPR:(k + 1) * _WPR] << k)
        pk_ref[pl.ds(2 * _LEVELS, _LEVELS), :] = word
        wrap = word[:, _WPR - 2:]
        wrap = lax.shift_left(wrap, 1) | lax.shift_right_logical(wrap, 31)
        pk_ref[pl.ds(0, _LEVELS), :] = jnp.concatenate(
            [wrap, word[:, :_WPR - 2]], axis=1)
        pk_ref[pl.ds(_LEVELS, _LEVELS), :] = jnp.concatenate(
            [wrap[:, 1:], word[:, :_WPR - 1]], axis=1)

        xv = x_ref[...]
        q = jnp.clip(jnp.round(xv * (_LEVELS - 1)).astype(jnp.int32),
                     0, _LEVELS - 1)
        ch = lax.broadcasted_iota(jnp.int32, (_B, 3), 1)
        idx_ref[...] = q + ch * _LEVELS

    pk, idx = pl.pallas_call(
        body,
        out_shape=[
            jax.ShapeDtypeStruct((3 * _LEVELS, _WPR), jnp.int32),
            jax.ShapeDtypeStruct((_B, 3), jnp.int32),
        ],
    )(x, w)
    return pk, idx.reshape(-1)


def _sc_xor(pk_tab, idx_flat):
    mesh = plsc.VectorSubcoreMesh(
        core_axis_name="c", subcore_axis_name="s", num_cores=2, num_subcores=16)

    @functools.partial(
        pl.kernel,
        mesh=mesh,
        out_type=jax.ShapeDtypeStruct((_B, _WPR), jnp.int32),
        scratch_types=[
            pltpu.VMEM((_ROWS_PER_W * 3,), jnp.int32),
            pltpu.VMEM((4, 3 * _G, _WPR), jnp.int32),
            pltpu.VMEM((4, _G, _WPR), jnp.int32),
            pltpu.SemaphoreType.DMA,
            pltpu.SemaphoreType.DMA,
        ],
    )
    def k(pk_hbm, idx_hbm, px_hbm, idx_v, gbuf, obuf, gsem, osem):
        wid = lax.axis_index("s") * 2 + lax.axis_index("c")
        base = wid * _ROWS_PER_W
        pltpu.sync_copy(idx_hbm.at[pl.ds(base * 3, _ROWS_PER_W * 3)], idx_v)

        def start_gather(g):
            pltpu.async_copy(
                pk_hbm.at[idx_v.at[pl.ds(g * (3 * _G), 3 * _G)]],
                gbuf.at[lax.bitwise_and(g, 3)], gsem)

        for g0 in range(3):
            start_gather(g0)

        def group(g, carry):
            par = lax.bitwise_and(g, 3)
            pltpu.make_async_copy(
                pk_hbm.at[idx_v.at[pl.ds(0, 3 * _G)]],
                gbuf.at[0], gsem).wait()

            @pl.when(g + 3 < _NGROUPS)
            def _():
                start_gather(g + 3)

            @pl.when(g >= 4)
            def _():
                pltpu.make_async_copy(
                    obuf.at[0], px_hbm.at[pl.ds(base, _G)], osem).wait()

            def row(p, c2):
                def wordblk(wi, c3):
                    a = gbuf[par, 3 * p, pl.ds(wi * 16, 16)]
                    b = gbuf[par, 3 * p + 1, pl.ds(wi * 16, 16)]
                    d = gbuf[par, 3 * p + 2, pl.ds(wi * 16, 16)]
                    obuf[par, p, pl.ds(wi * 16, 16)] = lax.bitwise_xor(
                        lax.bitwise_xor(a, b), d)
                    return c3

                lax.fori_loop(0, _WPR // 16, wordblk, 0)
                return c2

            lax.fori_loop(0, _G, row, 0)
            pltpu.async_copy(
                obuf.at[par], px_hbm.at[pl.ds(base + g * _G, _G)], osem)
            return carry

        lax.fori_loop(0, _NGROUPS, group, 0)
        for _ in range(4):
            pltpu.make_async_copy(
                obuf.at[0], px_hbm.at[pl.ds(base, _G)], osem).wait()

    return k(pk_tab, idx_flat)


def _tc_expand(px):
    sign_mask = np.int32(-2147483648)
    one_bits = np.int32(0x3F800000)

    def body(px_ref, o_ref):
        words = px_ref[...]
        for k in range(32):
            s = lax.bitwise_and(lax.shift_left(words, 31 - k), sign_mask)
            o_ref[:, pl.ds(k * _WPR, _WPR)] = lax.bitcast_convert_type(
                lax.bitwise_or(s, one_bits), jnp.float32)

    return pl.pallas_call(
        body,
        grid=(_B // _EXP_ROWS,),
        in_specs=[pl.BlockSpec((_EXP_ROWS, _WPR), lambda i: (i, 0))],
        out_specs=pl.BlockSpec((_EXP_ROWS, _D), lambda i: (i, 0)),
        out_shape=jax.ShapeDtypeStruct((_B, _D), jnp.float32),
    )(px)


def kernel(x, level_weight):
    pk_tab, idx_flat = _prep(x, level_weight)
    px = _sc_xor(pk_tab, idx_flat)
    return _tc_expand(px)
